# baseline (speedup 1.0000x reference)
import functools

import jax
import jax.numpy as jnp
from jax import lax
from jax.experimental import pallas as pl
from jax.experimental.pallas import tpu as pltpu
from jax.experimental.pallas import tpu_sc as plsc

N_NODES = 10000
N_EDGES = 160000
D = 256
DH = 128
EPS = 1e-5

NC = 2
NS = 16
EDGES_PER_TILE = N_EDGES // NS
CH_E = 2000
N_STAGE = EDGES_PER_TILE // CH_E
CHUNK = 80
NCHUNK = CH_E // CHUNK
NRING = 4
ROWS_PER_TILE = 640
TAIL_ROWS = N_NODES - 15 * ROWS_PER_TILE


_sc_mesh = plsc.VectorSubcoreMesh(
    core_axis_name="c", subcore_axis_name="s", num_cores=NC, num_subcores=NS
)


@functools.partial(
    pl.kernel,
    out_type=[jax.ShapeDtypeStruct((N_NODES, DH), jnp.float32),
              jax.ShapeDtypeStruct((N_NODES, DH), jnp.float32)],
    mesh=_sc_mesh,
    scratch_types=[
        pltpu.VMEM((CH_E,), jnp.int32),
        pltpu.VMEM((CH_E,), jnp.int32),
        [pltpu.VMEM((CHUNK, DH), jnp.float32) for _ in range(NRING)],
        pltpu.VMEM_SHARED((N_NODES, DH), jnp.float32),
        [pltpu.SemaphoreType.DMA for _ in range(NRING)],
        [pltpu.SemaphoreType.DMA for _ in range(2)],
    ],
)
def _sc_agg(x_hbm, src_hbm, dst_hbm, zeros_hbm, out_lo, out_hi,
            src_c, dst_c, rows, agg_sh, gsem, ssem):
    c = lax.axis_index("c")
    s = lax.axis_index("s")

    col = pl.ds(c * DH, DH)

    @pl.when(s < 15)
    def _():
        pltpu.sync_copy(zeros_hbm.at[s],
                        agg_sh.at[pl.ds(s * ROWS_PER_TILE, ROWS_PER_TILE)])

    @pl.when(s == 15)
    def _():
        pltpu.sync_copy(zeros_hbm.at[15].at[pl.ds(0, TAIL_ROWS)],
                        agg_sh.at[pl.ds(15 * ROWS_PER_TILE, TAIL_ROWS)])

    plsc.subcore_barrier()

    def _issue(off, i):
        return pltpu.async_copy(
            x_hbm.at[src_c.at[pl.ds(off, CHUNK)], col], rows[i], gsem[i])

    def _wait(i):
        pltpu.make_async_copy(
            x_hbm.at[src_c.at[pl.ds(0, CHUNK)], col], rows[i], gsem[i]).wait()

    def _scatter(off, i):
        pltpu.sync_copy(rows[i],
                        agg_sh.at[dst_c.at[pl.ds(off, CHUNK)]], add=True)

    def _stage(st, _):
        eb = s * EDGES_PER_TILE + st * CH_E
        pltpu.sync_copy(src_hbm.at[pl.ds(eb, CH_E)], src_c)
        pltpu.sync_copy(dst_hbm.at[pl.ds(eb, CH_E)], dst_c)

        for i in range(NRING):
            _issue(i * CHUNK, i)

        def _quad(j, _):
            for i in range(NRING):
                ch = j * NRING + i
                _wait(i)
                _scatter(ch * CHUNK, i)

                @pl.when(ch + NRING < NCHUNK)
                def _():
                    _issue((ch + NRING) * CHUNK, i)
            return 0

        lax.fori_loop(0, NCHUNK // NRING, _quad, 0)

        _wait(0)
        _scatter((NCHUNK - 1) * CHUNK, 0)
        return 0

    lax.fori_loop(0, N_STAGE, _stage, 0)

    plsc.subcore_barrier()

    def _writeout(dst_hbm_ref):
        @pl.when(s < 15)
        def _():
            own = pl.ds(s * ROWS_PER_TILE, ROWS_PER_TILE)
            pltpu.sync_copy(agg_sh.at[own], dst_hbm_ref.at[own])

        @pl.when(s == 15)
        def _():
            own = pl.ds(15 * ROWS_PER_TILE, TAIL_ROWS)
            pltpu.sync_copy(agg_sh.at[own], dst_hbm_ref.at[own])

    @pl.when(c == 0)
    def _():
        _writeout(out_lo)

    @pl.when(c == 1)
    def _():
        _writeout(out_hi)


def _tc_body(x_ref, aggl_ref, aggh_ref, wr_ref, wo_ref, bnw_ref, bnb_ref, o_ref):
    r = lax.dot_general(
        x_ref[...], wo_ref[...], (((1,), (1,)), ((), ())),
        preferred_element_type=jnp.float32, precision=lax.Precision.DEFAULT,
    )
    r = r + lax.dot_general(
        aggl_ref[...], wr_ref[:, :DH], (((1,), (1,)), ((), ())),
        preferred_element_type=jnp.float32, precision=lax.Precision.DEFAULT,
    )
    r = r + lax.dot_general(
        aggh_ref[...], wr_ref[:, DH:], (((1,), (1,)), ((), ())),
        preferred_element_type=jnp.float32, precision=lax.Precision.DEFAULT,
    )
    mean = jnp.sum(r, axis=0) / N_NODES
    var = jnp.sum(r * r, axis=0) / N_NODES - mean * mean
    scale = bnw_ref[0, :] * lax.rsqrt(var + EPS)
    shift = bnb_ref[0, :] - mean * scale
    y = r * scale[None, :] + shift[None, :]
    o_ref[...] = jnp.where(y >= 0, y, 0.1 * y)


def kernel(x, edge_index, W_rel, b_rel, W_root, bn_weight, bn_bias):
    del b_rel
    src = edge_index[0].astype(jnp.int32)
    dst = edge_index[1].astype(jnp.int32)
    zeros = jnp.zeros((NS, ROWS_PER_TILE, DH), jnp.float32)

    agg_lo, agg_hi = _sc_agg(x, src, dst, zeros)

    out = pl.pallas_call(
        _tc_body,
        in_specs=[
            pl.BlockSpec((N_NODES, D), lambda: (0, 0)),
            pl.BlockSpec((N_NODES, DH), lambda: (0, 0)),
            pl.BlockSpec((N_NODES, DH), lambda: (0, 0)),
            pl.BlockSpec((D, D), lambda: (0, 0)),
            pl.BlockSpec((D, D), lambda: (0, 0)),
            pl.BlockSpec((1, D), lambda: (0, 0)),
            pl.BlockSpec((1, D), lambda: (0, 0)),
        ],
        out_specs=pl.BlockSpec((N_NODES, D), lambda: (0, 0)),
        out_shape=jax.ShapeDtypeStruct((N_NODES, D), jnp.float32),
    )(x, agg_lo, agg_hi, W_rel, W_root,
      bn_weight.reshape(1, D), bn_bias.reshape(1, D))

    return out


# speedup vs baseline: 1.0011x; 1.0011x over previous
import functools

import jax
import jax.numpy as jnp
from jax import lax
from jax.experimental import pallas as pl
from jax.experimental.pallas import tpu as pltpu
from jax.experimental.pallas import tpu_sc as plsc

N_NODES = 10000
N_EDGES = 160000
D = 256
DH = 128
EPS = 1e-5

NC = 2
NS = 16
EDGES_PER_TILE = N_EDGES // NS
CH_E = 2000
N_STAGE = EDGES_PER_TILE // CH_E
CHUNK = 80
NCHUNK = CH_E // CHUNK
NRING = 4
ROWS_PER_TILE = 640
TAIL_ROWS = N_NODES - 15 * ROWS_PER_TILE


_sc_mesh = plsc.VectorSubcoreMesh(
    core_axis_name="c", subcore_axis_name="s", num_cores=NC, num_subcores=NS
)


@functools.partial(
    pl.kernel,
    out_type=[jax.ShapeDtypeStruct((N_NODES, DH), jnp.float32),
              jax.ShapeDtypeStruct((N_NODES, DH), jnp.float32)],
    mesh=_sc_mesh,
    scratch_types=[
        pltpu.VMEM((CH_E,), jnp.int32),
        pltpu.VMEM((CH_E,), jnp.int32),
        [pltpu.VMEM((CHUNK, DH), jnp.float32) for _ in range(NRING)],
        pltpu.VMEM_SHARED((N_NODES, DH), jnp.float32),
        [pltpu.SemaphoreType.DMA for _ in range(NRING)],
        [pltpu.SemaphoreType.DMA for _ in range(2)],
    ],
)
def _sc_agg(x_hbm, src_hbm, dst_hbm, zeros_hbm, out_lo, out_hi,
            src_c, dst_c, rows, agg_sh, gsem, ssem):
    c = lax.axis_index("c")
    s = lax.axis_index("s")

    col = pl.ds(c * DH, DH)

    @pl.when(s < 15)
    def _():
        pltpu.sync_copy(zeros_hbm,
                        agg_sh.at[pl.ds(s * ROWS_PER_TILE, ROWS_PER_TILE)])

    @pl.when(s == 15)
    def _():
        pltpu.sync_copy(zeros_hbm.at[pl.ds(0, TAIL_ROWS)],
                        agg_sh.at[pl.ds(15 * ROWS_PER_TILE, TAIL_ROWS)])

    plsc.subcore_barrier()

    def _issue(off, i):
        return pltpu.async_copy(
            x_hbm.at[src_c.at[pl.ds(off, CHUNK)], col], rows[i], gsem[i])

    def _wait(i):
        pltpu.make_async_copy(
            x_hbm.at[src_c.at[pl.ds(0, CHUNK)], col], rows[i], gsem[i]).wait()

    def _scatter(off, i):
        pltpu.sync_copy(rows[i],
                        agg_sh.at[dst_c.at[pl.ds(off, CHUNK)]], add=True)

    def _stage(st, _):
        eb = s * EDGES_PER_TILE + st * CH_E
        pltpu.sync_copy(src_hbm.at[pl.ds(eb, CH_E)], src_c)
        pltpu.sync_copy(dst_hbm.at[pl.ds(eb, CH_E)], dst_c)

        for i in range(NRING):
            _issue(i * CHUNK, i)

        def _quad(j, _):
            for i in range(NRING):
                ch = j * NRING + i
                _wait(i)
                _scatter(ch * CHUNK, i)

                @pl.when(ch + NRING < NCHUNK)
                def _():
                    _issue((ch + NRING) * CHUNK, i)
            return 0

        lax.fori_loop(0, NCHUNK // NRING, _quad, 0)

        _wait(0)
        _scatter((NCHUNK - 1) * CHUNK, 0)
        return 0

    lax.fori_loop(0, N_STAGE, _stage, 0)

    plsc.subcore_barrier()

    def _writeout(dst_hbm_ref):
        @pl.when(s < 15)
        def _():
            own = pl.ds(s * ROWS_PER_TILE, ROWS_PER_TILE)
            pltpu.sync_copy(agg_sh.at[own], dst_hbm_ref.at[own])

        @pl.when(s == 15)
        def _():
            own = pl.ds(15 * ROWS_PER_TILE, TAIL_ROWS)
            pltpu.sync_copy(agg_sh.at[own], dst_hbm_ref.at[own])

    @pl.when(c == 0)
    def _():
        _writeout(out_lo)

    @pl.when(c == 1)
    def _():
        _writeout(out_hi)


def _tc_body(x_ref, aggl_ref, aggh_ref, wr_ref, wo_ref, bnw_ref, bnb_ref, o_ref):
    r = lax.dot_general(
        x_ref[...], wo_ref[...], (((1,), (1,)), ((), ())),
        preferred_element_type=jnp.float32, precision=lax.Precision.DEFAULT,
    )
    r = r + lax.dot_general(
        aggl_ref[...], wr_ref[:, :DH], (((1,), (1,)), ((), ())),
        preferred_element_type=jnp.float32, precision=lax.Precision.DEFAULT,
    )
    r = r + lax.dot_general(
        aggh_ref[...], wr_ref[:, DH:], (((1,), (1,)), ((), ())),
        preferred_element_type=jnp.float32, precision=lax.Precision.DEFAULT,
    )
    mean = jnp.sum(r, axis=0) / N_NODES
    var = jnp.sum(r * r, axis=0) / N_NODES - mean * mean
    scale = bnw_ref[0, :] * lax.rsqrt(var + EPS)
    shift = bnb_ref[0, :] - mean * scale
    y = r * scale[None, :] + shift[None, :]
    o_ref[...] = jnp.where(y >= 0, y, 0.1 * y)


def kernel(x, edge_index, W_rel, b_rel, W_root, bn_weight, bn_bias):
    del b_rel
    src = edge_index[0].astype(jnp.int32)
    dst = edge_index[1].astype(jnp.int32)
    zeros = jnp.zeros((ROWS_PER_TILE, DH), jnp.float32)

    agg_lo, agg_hi = _sc_agg(x, src, dst, zeros)

    out = pl.pallas_call(
        _tc_body,
        in_specs=[
            pl.BlockSpec((N_NODES, D), lambda: (0, 0)),
            pl.BlockSpec((N_NODES, DH), lambda: (0, 0)),
            pl.BlockSpec((N_NODES, DH), lambda: (0, 0)),
            pl.BlockSpec((D, D), lambda: (0, 0)),
            pl.BlockSpec((D, D), lambda: (0, 0)),
            pl.BlockSpec((1, D), lambda: (0, 0)),
            pl.BlockSpec((1, D), lambda: (0, 0)),
        ],
        out_specs=pl.BlockSpec((N_NODES, D), lambda: (0, 0)),
        out_shape=jax.ShapeDtypeStruct((N_NODES, D), jnp.float32),
    )(x, agg_lo, agg_hi, W_rel, W_root,
      bn_weight.reshape(1, D), bn_bias.reshape(1, D))

    return out
